# baseline (speedup 1.0000x reference)
import functools

import jax
import jax.numpy as jnp
from jax import lax
from jax.experimental import pallas as pl
from jax.experimental.pallas import tpu as pltpu
from jax.experimental.pallas import tpu_sc as plsc

ACT_DIM = 2048
DICT_SIZE = 32768
TOPK = 64
BATCH = 32

F_TILE = 8192
K_TILE = 512
NF = DICT_SIZE // F_TILE
NK = ACT_DIM // K_TILE

SIGN = -(2**31)
NC = 2
NS = 16
L = 16


def _lane_bcast(v, j):
    idx = jnp.full((L, 1), j, jnp.int32)
    dnums = lax.GatherDimensionNumbers(
        offset_dims=(), collapsed_slice_dims=(0,), start_index_map=(0,))
    return lax.gather(v, idx, dnums, slice_sizes=(1,),
                      mode=lax.GatherScatterMode.PROMISE_IN_BOUNDS)


def _sortable_key(bits):
    return jnp.where(bits >= 0, bits, bits ^ jnp.int32(0x7FFFFFFF))


BLK = 128
N_BLK = DICT_SIZE // BLK


def _tc_encode_body(x_ref, bdec_ref, w_ref, benc_ref,
                    pre_ref, thr_ref, ecnt_ref, blk_ref, keys_ref):
    f = pl.program_id(0)
    k = pl.program_id(1)
    xm = x_ref[...] - bdec_ref[...]
    part = jnp.dot(xm, w_ref[...], preferred_element_type=jnp.float32)

    @pl.when(k == 0)
    def _init():
        pre_ref[...] = part + benc_ref[...]

    @pl.when(k > 0)
    def _acc():
        pre_ref[...] += part

    @pl.when(k == NK - 1)
    def _keys():
        bits = lax.bitcast_convert_type(pre_ref[...], jnp.int32)
        keys_ref[:, pl.ds(f * F_TILE, F_TILE)] = _sortable_key(bits)

    @pl.when((f == NF - 1) & (k == NK - 1))
    def _finalize():
        keys = keys_ref[...]

        def search_cond(carry):
            it, _, frozen = carry
            return (it < 32) & (jnp.min(frozen) == 0)

        def bit_step(carry):
            it, t_u, frozen = carry
            cand_u = t_u | (jnp.int32(1) << (31 - it))
            cand_s = cand_u ^ jnp.int32(SIGN)
            cnt = jnp.sum((keys >= cand_s).astype(jnp.int32), axis=1,
                          keepdims=True)
            take = (frozen == 0) & (cnt >= TOPK)
            t_u = jnp.where(take, cand_u, t_u)
            frozen = jnp.where(take & (cnt == TOPK), jnp.int32(1), frozen)
            return it + 1, t_u, frozen

        _, t_u, _ = lax.while_loop(
            search_cond, bit_step,
            (jnp.int32(0), jnp.zeros((BATCH, 1), jnp.int32),
             jnp.zeros((BATCH, 1), jnp.int32)))
        thr_s = t_u ^ jnp.int32(SIGN)
        c_gt = jnp.sum((keys > thr_s).astype(jnp.int32), axis=1,
                       keepdims=True)
        c_ge = jnp.sum((keys >= thr_s).astype(jnp.int32), axis=1,
                       keepdims=True)
        ecnt = jnp.int32(TOPK) - c_gt
        simp = (c_ge == TOPK).astype(jnp.int32)
        thr_ref[...] = jnp.broadcast_to(thr_s, (BATCH, 128))
        ecnt_ref[...] = jnp.broadcast_to(ecnt * (1 - simp) - simp,
                                         (BATCH, 128))
        blk_of = lax.broadcasted_iota(jnp.int32, (2048, 2048 // BLK), 0)
        col_of = lax.broadcasted_iota(jnp.int32, (2048, 2048 // BLK), 1)
        bsum = (blk_of // BLK == col_of).astype(jnp.float32)
        parts = []
        cw = 2048
        for i2 in range(DICT_SIZE // cw):
            m = (keys[:, i2 * cw:(i2 + 1) * cw] >= thr_s)
            parts.append(jnp.dot(m.astype(jnp.float32), bsum,
                                 preferred_element_type=jnp.float32))
        blk_ref[...] = jnp.concatenate(parts, axis=1).astype(jnp.int32)


@jax.jit
def _tc_encode(x, W_enc, b_enc, b_dec):
    benc2 = b_enc.reshape(1, DICT_SIZE)
    bdec2 = b_dec.reshape(1, ACT_DIM)
    return pl.pallas_call(
        _tc_encode_body,
        grid=(NF, NK),
        in_specs=[
            pl.BlockSpec((BATCH, K_TILE), lambda f, k: (0, k)),
            pl.BlockSpec((1, K_TILE), lambda f, k: (0, k)),
            pl.BlockSpec((K_TILE, F_TILE), lambda f, k: (k, f)),
            pl.BlockSpec((1, F_TILE), lambda f, k: (0, f)),
        ],
        out_specs=[
            pl.BlockSpec((BATCH, F_TILE), lambda f, k: (0, f)),
            pl.BlockSpec((BATCH, 128), lambda f, k: (0, 0)),
            pl.BlockSpec((BATCH, 128), lambda f, k: (0, 0)),
            pl.BlockSpec((BATCH, N_BLK), lambda f, k: (0, 0)),
        ],
        out_shape=[
            jax.ShapeDtypeStruct((BATCH, DICT_SIZE), jnp.float32),
            jax.ShapeDtypeStruct((BATCH, 128), jnp.int32),
            jax.ShapeDtypeStruct((BATCH, 128), jnp.int32),
            jax.ShapeDtypeStruct((BATCH, N_BLK), jnp.int32),
        ],
        scratch_shapes=[pltpu.VMEM((BATCH, DICT_SIZE), jnp.int32)],
    )(x, bdec2, W_enc, benc2)


def _sc_decode_body(pre_hbm, thr_hbm, ecnt_hbm, blk_hbm, wdec_hbm, bdec_hbm,
                    out_hbm, row_v, meta_v, flags_v, idx_v, val_v,
                    rows_a, rows_b, acc_v, sem_a, sem_b, sem_c):
    wid = lax.axis_index("s") * NC + lax.axis_index("c")

    pltpu.sync_copy(pre_hbm.at[wid], row_v)
    pltpu.sync_copy(blk_hbm.at[wid], flags_v.at[pl.ds(0, N_BLK)])
    pltpu.sync_copy(thr_hbm.at[wid, pl.ds(0, L)], meta_v.at[pl.ds(0, L)])
    pltpu.sync_copy(ecnt_hbm.at[wid, pl.ds(0, L)], meta_v.at[pl.ds(L, L)])
    t_b = meta_v[pl.ds(0, L)]
    e_b = meta_v[pl.ds(L, L)]

    def load_chunk(g):
        return row_v[g // (ACT_DIM // L), pl.ds((g % (ACT_DIM // L)) * L, L)]

    def emit(pos, sel, vals, g):
        dst = pos + plsc.cumsum(sel.astype(jnp.int32)) - 1
        idxs = g * L + lax.iota(jnp.int32, L)
        plsc.store_scatter(idx_v, [dst], idxs, mask=sel)
        plsc.store_scatter(val_v, [dst], vals, mask=sel)
        return pos + jnp.sum(sel.astype(jnp.int32))

    def chunk_full(g, pos, eqs):
        vals = load_chunk(g)
        key = _sortable_key(lax.bitcast_convert_type(vals, jnp.int32))
        m_gt = key > t_b
        m_eq = key == t_b
        prefix_eq = plsc.cumsum(m_eq.astype(jnp.int32))
        sel = m_gt | (m_eq & ((eqs + prefix_eq) <= e_b))
        pos = emit(pos, sel, vals, g)
        eqs = eqs + jnp.sum(m_eq.astype(jnp.int32))
        return pos, eqs

    def chunk_fast(g, pos):
        vals = load_chunk(g)
        key = _sortable_key(lax.bitcast_convert_type(vals, jnp.int32))
        return emit(pos, key >= t_b, vals, g)

    def scan_full(_):
        def block_step(blk, carry):
            def process(carry):
                pos, eqs = carry
                for k in range(BLK // L):
                    pos, eqs = chunk_full(blk * (BLK // L) + k, pos, eqs)
                return pos, eqs

            flag = flags_v[pl.ds(blk, L)][0]
            return lax.cond(flag > 0, process, lambda c: c, carry)

        lax.fori_loop(0, N_BLK, block_step, (jnp.int32(0), jnp.int32(0)))
        return 0

    def scan_fast(_):
        def block_step(blk, pos):
            def process(pos):
                for k in range(BLK // L):
                    pos = chunk_fast(blk * (BLK // L) + k, pos)
                return pos

            flag = flags_v[pl.ds(blk, L)][0]
            return lax.cond(flag > 0, process, lambda p: p, pos)

        lax.fori_loop(0, N_BLK, block_step, jnp.int32(0))
        return 0

    lax.cond(e_b[0] < 0, scan_fast, scan_full, 0)

    pltpu.sync_copy(bdec_hbm, acc_v)

    n_grp = TOPK // L
    bufs = [rows_a, rows_b, row_v]
    sems = [sem_a, sem_b, sem_c]
    pend = [None] * n_grp
    for g in range(3):
        pend[g] = pltpu.async_copy(wdec_hbm.at[idx_v[pl.ds(g * L, L)]],
                                   bufs[g], sems[g])
    for grp in range(n_grp):
        pend[grp].wait()
        rows_v = bufs[grp % 3]
        gvals = val_v[pl.ds(grp * L, L)]
        wj = [_lane_bcast(gvals, j) for j in range(L)]

        def acc_step(c, _, wj=wj, rows_v=rows_v):
            for u in range(2):
                cc = c * 2 + u
                a = acc_v[pl.ds(cc * L, L)]
                for j in range(L):
                    a = a + wj[j] * rows_v[j, pl.ds(cc * L, L)]
                acc_v[pl.ds(cc * L, L)] = a
            return 0

        lax.fori_loop(0, ACT_DIM // (2 * L), acc_step, 0)
        if grp + 3 < n_grp:
            nxt = idx_v[pl.ds((grp + 3) * L, L)]
            pend[grp + 3] = pltpu.async_copy(
                wdec_hbm.at[nxt], bufs[(grp + 3) % 3], sems[(grp + 3) % 3])

    pltpu.sync_copy(acc_v, out_hbm.at[wid])


@jax.jit
def _sc_decode(pre, thr, ecnt, blk, W_dec, b_dec):
    mesh = plsc.VectorSubcoreMesh(core_axis_name="c", subcore_axis_name="s")
    return pl.kernel(
        _sc_decode_body,
        out_type=jax.ShapeDtypeStruct((BATCH, ACT_DIM), jnp.float32),
        mesh=mesh,
        compiler_params=pltpu.CompilerParams(needs_layout_passes=False),
        scratch_types=[
            pltpu.VMEM((L, ACT_DIM), jnp.float32),
            pltpu.VMEM((2 * L,), jnp.int32),
            pltpu.VMEM((N_BLK + L,), jnp.int32),
            pltpu.VMEM((TOPK + L,), jnp.int32),
            pltpu.VMEM((TOPK + L,), jnp.float32),
            pltpu.VMEM((L, ACT_DIM), jnp.float32),
            pltpu.VMEM((L, ACT_DIM), jnp.float32),
            pltpu.VMEM((ACT_DIM,), jnp.float32),
            pltpu.SemaphoreType.DMA,
            pltpu.SemaphoreType.DMA,
            pltpu.SemaphoreType.DMA,
        ],
    )(pre.reshape(BATCH, L, ACT_DIM), thr, ecnt, blk, W_dec, b_dec)


def kernel(x, W_enc, b_enc, W_dec, b_dec):
    pre, thr, ecnt, blk = _tc_encode(x, W_enc, b_enc, b_dec)
    return _sc_decode(pre, thr, ecnt, blk, W_dec, b_dec)


# speedup vs baseline: 1.0578x; 1.0578x over previous
import functools

import jax
import jax.numpy as jnp
from jax import lax
from jax.experimental import pallas as pl
from jax.experimental.pallas import tpu as pltpu
from jax.experimental.pallas import tpu_sc as plsc

ACT_DIM = 2048
DICT_SIZE = 32768
TOPK = 64
BATCH = 32

F_TILE = 8192
K_TILE = 512
NF = DICT_SIZE // F_TILE
NK = ACT_DIM // K_TILE

SIGN = -(2**31)
NC = 2
NS = 16
L = 16


def _lane_bcast(v, j):
    idx = jnp.full((L, 1), j, jnp.int32)
    dnums = lax.GatherDimensionNumbers(
        offset_dims=(), collapsed_slice_dims=(0,), start_index_map=(0,))
    return lax.gather(v, idx, dnums, slice_sizes=(1,),
                      mode=lax.GatherScatterMode.PROMISE_IN_BOUNDS)


def _sortable_key(bits):
    return jnp.where(bits >= 0, bits, bits ^ jnp.int32(0x7FFFFFFF))


BLK = 128
N_BLK = DICT_SIZE // BLK


def _tc_encode_body(x_ref, bdec_ref, w_ref, benc_ref,
                    pre_ref, thr_ref, ecnt_ref, blk_ref, keys_ref):
    f = pl.program_id(0)
    k = pl.program_id(1)
    xm = x_ref[...] - bdec_ref[...]
    part = jnp.dot(xm, w_ref[...], preferred_element_type=jnp.float32)

    @pl.when(k == 0)
    def _init():
        pre_ref[...] = part + benc_ref[...]

    @pl.when(k > 0)
    def _acc():
        pre_ref[...] += part

    @pl.when(k == NK - 1)
    def _keys():
        bits = lax.bitcast_convert_type(pre_ref[...], jnp.int32)
        keys_ref[:, pl.ds(f * F_TILE, F_TILE)] = _sortable_key(bits)

    @pl.when((f == NF - 1) & (k == NK - 1))
    def _finalize():
        keys = keys_ref[...]

        def search_cond(carry):
            it, _, frozen = carry
            return (it < 32) & (jnp.min(frozen) == 0)

        def bit_step(carry):
            it, t_u, frozen = carry
            cand_u = t_u | (jnp.int32(1) << (31 - it))
            cand_s = cand_u ^ jnp.int32(SIGN)
            cnt = jnp.sum((keys >= cand_s).astype(jnp.int32), axis=1,
                          keepdims=True)
            take = (frozen == 0) & (cnt >= TOPK)
            t_u = jnp.where(take, cand_u, t_u)
            frozen = jnp.where(take & (cnt == TOPK), jnp.int32(1), frozen)
            return it + 1, t_u, frozen

        _, t_u, _ = lax.while_loop(
            search_cond, bit_step,
            (jnp.int32(0), jnp.zeros((BATCH, 1), jnp.int32),
             jnp.zeros((BATCH, 1), jnp.int32)))
        thr_s = t_u ^ jnp.int32(SIGN)
        c_gt = jnp.sum((keys > thr_s).astype(jnp.int32), axis=1,
                       keepdims=True)
        c_ge = jnp.sum((keys >= thr_s).astype(jnp.int32), axis=1,
                       keepdims=True)
        ecnt = jnp.int32(TOPK) - c_gt
        simp = (c_ge == TOPK).astype(jnp.int32)
        thr_ref[...] = jnp.broadcast_to(thr_s, (BATCH, 128))
        ecnt_ref[...] = jnp.broadcast_to(ecnt * (1 - simp) - simp,
                                         (BATCH, 128))
        blk_of = lax.broadcasted_iota(jnp.int32, (2048, 2048 // BLK), 0)
        col_of = lax.broadcasted_iota(jnp.int32, (2048, 2048 // BLK), 1)
        bsum = (blk_of // BLK == col_of).astype(jnp.float32)
        parts = []
        cw = 2048
        for i2 in range(DICT_SIZE // cw):
            m = (keys[:, i2 * cw:(i2 + 1) * cw] >= thr_s)
            parts.append(jnp.dot(m.astype(jnp.float32), bsum,
                                 preferred_element_type=jnp.float32))
        blk_ref[...] = jnp.concatenate(parts, axis=1).astype(jnp.int32)


@jax.jit
def _tc_encode(x, W_enc, b_enc, b_dec):
    benc2 = b_enc.reshape(1, DICT_SIZE)
    bdec2 = b_dec.reshape(1, ACT_DIM)
    return pl.pallas_call(
        _tc_encode_body,
        grid=(NF, NK),
        in_specs=[
            pl.BlockSpec((BATCH, K_TILE), lambda f, k: (0, k)),
            pl.BlockSpec((1, K_TILE), lambda f, k: (0, k)),
            pl.BlockSpec((K_TILE, F_TILE), lambda f, k: (k, f)),
            pl.BlockSpec((1, F_TILE), lambda f, k: (0, f)),
        ],
        out_specs=[
            pl.BlockSpec((BATCH, F_TILE), lambda f, k: (0, f)),
            pl.BlockSpec((BATCH, 128), lambda f, k: (0, 0)),
            pl.BlockSpec((BATCH, 128), lambda f, k: (0, 0)),
            pl.BlockSpec((BATCH, N_BLK), lambda f, k: (0, 0)),
        ],
        out_shape=[
            jax.ShapeDtypeStruct((BATCH, DICT_SIZE), jnp.float32),
            jax.ShapeDtypeStruct((BATCH, 128), jnp.int32),
            jax.ShapeDtypeStruct((BATCH, 128), jnp.int32),
            jax.ShapeDtypeStruct((BATCH, N_BLK), jnp.int32),
        ],
        scratch_shapes=[pltpu.VMEM((BATCH, DICT_SIZE), jnp.int32)],
    )(x, bdec2, W_enc, benc2)


def _sc_decode_body(pre_hbm, thr_hbm, ecnt_hbm, blk_hbm, wdec_hbm, bdec_hbm,
                    out_hbm, row_v, meta_v, flags_v, idx_v, val_v,
                    rows_a, rows_b, acc_v, sem_a, sem_b):
    wid = lax.axis_index("s") * NC + lax.axis_index("c")

    pltpu.sync_copy(pre_hbm.at[wid], row_v)
    pltpu.sync_copy(blk_hbm.at[wid], flags_v.at[pl.ds(0, N_BLK)])
    pltpu.sync_copy(thr_hbm.at[wid, pl.ds(0, L)], meta_v.at[pl.ds(0, L)])
    pltpu.sync_copy(ecnt_hbm.at[wid, pl.ds(0, L)], meta_v.at[pl.ds(L, L)])
    t_b = meta_v[pl.ds(0, L)]
    e_b = meta_v[pl.ds(L, L)]

    def emit(pos, sel, vals, g):
        dst = pos + plsc.cumsum(sel.astype(jnp.int32)) - 1
        idxs = g * L + lax.iota(jnp.int32, L)
        plsc.store_scatter(idx_v, [dst], idxs, mask=sel)
        plsc.store_scatter(val_v, [dst], vals, mask=sel)
        return pos + jnp.sum(sel.astype(jnp.int32))

    def chunk_sel(g, pos, eqs):
        vals = row_v[pl.ds(g * L, L)]
        key = _sortable_key(lax.bitcast_convert_type(vals, jnp.int32))
        m_gt = key > t_b
        m_eq = key == t_b
        prefix_eq = plsc.cumsum(m_eq.astype(jnp.int32))
        sel = m_gt | (m_eq & ((eqs + prefix_eq) <= e_b))
        pos = emit(pos, sel, vals, g)
        eqs = eqs + jnp.sum(m_eq.astype(jnp.int32))
        return pos, eqs

    def chunk_fast(g, pos):
        vals = row_v[pl.ds(g * L, L)]
        key = _sortable_key(lax.bitcast_convert_type(vals, jnp.int32))
        return emit(pos, key >= t_b, vals, g)

    def scan_full(_):
        def block_step(blk, carry):
            def process(carry):
                pos, eqs = carry
                for k in range(BLK // L):
                    pos, eqs = chunk_sel(blk * (BLK // L) + k, pos, eqs)
                return pos, eqs

            flag = flags_v[pl.ds(blk, L)][0]
            return lax.cond(flag > 0, process, lambda c: c, carry)

        lax.fori_loop(0, N_BLK, block_step, (jnp.int32(0), jnp.int32(0)))
        return 0

    def scan_fast(_):
        def block_step(blk, pos):
            def process(pos):
                for k in range(BLK // L):
                    pos = chunk_fast(blk * (BLK // L) + k, pos)
                return pos

            flag = flags_v[pl.ds(blk, L)][0]
            return lax.cond(flag > 0, process, lambda p: p, pos)

        lax.fori_loop(0, N_BLK, block_step, jnp.int32(0))
        return 0

    lax.cond(e_b[0] < 0, scan_fast, scan_full, 0)

    pltpu.sync_copy(bdec_hbm, acc_v)

    n_grp = TOPK // L
    bufs = [rows_a, rows_b]
    sems = [sem_a, sem_b]
    pend = [None] * n_grp
    pend[0] = pltpu.async_copy(wdec_hbm.at[idx_v[pl.ds(0, L)]],
                               bufs[0], sems[0])
    for grp in range(n_grp):
        if grp + 1 < n_grp:
            nxt = idx_v[pl.ds((grp + 1) * L, L)]
            pend[grp + 1] = pltpu.async_copy(
                wdec_hbm.at[nxt], bufs[(grp + 1) % 2], sems[(grp + 1) % 2])
        pend[grp].wait()
        rows_v = bufs[grp % 2]
        gvals = val_v[pl.ds(grp * L, L)]
        wj = [_lane_bcast(gvals, j) for j in range(L)]

        def acc_step(c, _, wj=wj, rows_v=rows_v):
            for u in range(2):
                cc = c * 2 + u
                a = acc_v[pl.ds(cc * L, L)]
                for j in range(L):
                    a = a + wj[j] * rows_v[j, pl.ds(cc * L, L)]
                acc_v[pl.ds(cc * L, L)] = a
            return 0

        lax.fori_loop(0, ACT_DIM // (2 * L), acc_step, 0)

    pltpu.sync_copy(acc_v, out_hbm.at[wid])


@jax.jit
def _sc_decode(pre, thr, ecnt, blk, W_dec, b_dec):
    mesh = plsc.VectorSubcoreMesh(core_axis_name="c", subcore_axis_name="s")
    return pl.kernel(
        _sc_decode_body,
        out_type=jax.ShapeDtypeStruct((BATCH, ACT_DIM), jnp.float32),
        mesh=mesh,
        compiler_params=pltpu.CompilerParams(needs_layout_passes=False),
        scratch_types=[
            pltpu.VMEM((DICT_SIZE,), jnp.float32),
            pltpu.VMEM((2 * L,), jnp.int32),
            pltpu.VMEM((N_BLK + L,), jnp.int32),
            pltpu.VMEM((TOPK + L,), jnp.int32),
            pltpu.VMEM((TOPK + L,), jnp.float32),
            pltpu.VMEM((L, ACT_DIM), jnp.float32),
            pltpu.VMEM((L, ACT_DIM), jnp.float32),
            pltpu.VMEM((ACT_DIM,), jnp.float32),
            pltpu.SemaphoreType.DMA,
            pltpu.SemaphoreType.DMA,
        ],
    )(pre, thr, ecnt, blk, W_dec, b_dec)


def kernel(x, W_enc, b_enc, W_dec, b_dec):
    pre, thr, ecnt, blk = _tc_encode(x, W_enc, b_enc, b_dec)
    return _sc_decode(pre, thr, ecnt, blk, W_dec, b_dec)
